# baseline (speedup 1.0000x reference)
import functools

import jax
import jax.numpy as jnp
from jax import lax
from jax.experimental import pallas as pl
from jax.experimental.pallas import tpu as pltpu
from jax.experimental.pallas import tpu_sc as plsc

B, S, D = 4096, 77, 512
SP = 80
NC, NS = 2, 16
NW = NC * NS
BPW = B // NW
LANES = 16


def _emb_body(x_hbm, tok_hbm, pos_hbm, out_hbm,
              pos_v, idx0, idx1, main0, main1,
              isem0, isem1, gsem0, gsem1, osem0, osem1):
    wid = lax.axis_index("s") * NC + lax.axis_index("c")
    b0 = wid * BPW

    pltpu.sync_copy(pos_hbm, pos_v)

    idxs = (idx0, idx1)
    mains = (main0, main1)
    isems = (isem0, isem1)
    gsems = (gsem0, gsem1)
    osems = (osem0, osem1)

    def idx_src(c):
        return x_hbm.at[pl.ds((b0 + c) * SP, SP)]

    def start_idx(c, slot):
        pltpu.async_copy(idx_src(c), idxs[slot], isems[slot])

    def wait_idx(c, slot):
        pltpu.make_async_copy(idx_src(c), idxs[slot], isems[slot]).wait()

    def start_gather(c, slot):
        pltpu.async_copy(tok_hbm.at[idxs[slot]], mains[slot], gsems[slot])

    def wait_gather(c, slot):
        pltpu.make_async_copy(tok_hbm.at[idxs[slot]], mains[slot], gsems[slot]).wait()

    def start_out(c, slot):
        pltpu.async_copy(mains[slot], out_hbm.at[pl.ds((b0 + c) * SP, SP)], osems[slot])

    def wait_out(c, slot):
        pltpu.make_async_copy(mains[slot], out_hbm.at[pl.ds((b0 + c) * SP, SP)], osems[slot]).wait()

    start_idx(0, 0)
    start_idx(1, 1)
    wait_idx(0, 0)
    start_gather(0, 0)

    def pair(p, carry):
        for sl_ in range(2):
            c = 2 * p + sl_
            slot, nslot = sl_, 1 - sl_

            @pl.when(c >= 1)
            def _():
                wait_out(c - 1, nslot)

            @pl.when(c + 1 < BPW)
            def _():
                wait_idx(c + 1, nslot)
                start_gather(c + 1, nslot)

            wait_gather(c, slot)

            @pl.when(c + 2 < BPW)
            def _():
                start_idx(c + 2, slot)

            main_v = mains[slot]

            @plsc.parallel_loop(0, S, unroll=4)
            def _(r):
                for j in range(D // LANES):
                    dsl = pl.ds(j * LANES, LANES)
                    main_v[r, dsl] = main_v[r, dsl] + pos_v[r, dsl]

            start_out(c, slot)
        return carry

    lax.fori_loop(0, BPW // 2, pair, 0)
    wait_out(BPW - 1, 1)


@functools.partial(
    pl.kernel,
    out_type=jax.ShapeDtypeStruct((B * SP, D), jnp.float32),
    mesh=plsc.VectorSubcoreMesh(
        core_axis_name="c", subcore_axis_name="s", num_cores=NC, num_subcores=NS
    ),
    scratch_types=[
        pltpu.VMEM((S, D), jnp.float32),
        pltpu.VMEM((SP,), jnp.int32),
        pltpu.VMEM((SP,), jnp.int32),
        pltpu.VMEM((SP, D), jnp.float32),
        pltpu.VMEM((SP, D), jnp.float32),
        pltpu.SemaphoreType.DMA,
        pltpu.SemaphoreType.DMA,
        pltpu.SemaphoreType.DMA,
        pltpu.SemaphoreType.DMA,
        pltpu.SemaphoreType.DMA,
        pltpu.SemaphoreType.DMA,
    ],
)
def _emb(x_hbm, tok_hbm, pos_hbm, out_hbm, *rest):
    _emb_body(x_hbm, tok_hbm, pos_hbm, out_hbm, *rest)


def kernel(x, token_table, position_table):
    x_pad = jnp.pad(x.astype(jnp.int32), ((0, 0), (0, SP - S))).reshape(B * SP)
    return _emb(x_pad, token_table, position_table).reshape(B, SP, D)[:, :S, :]


# speedup vs baseline: 1.0419x; 1.0419x over previous
import functools

import jax
import jax.numpy as jnp
from jax import lax
from jax.experimental import pallas as pl
from jax.experimental.pallas import tpu as pltpu
from jax.experimental.pallas import tpu_sc as plsc

B, S, D = 4096, 77, 512
SP = 80
MAIN = 72
TAIL = S - MAIN
NC, NS = 2, 16
NW = NC * NS
BPW = B // NW
LANES = 16


def _emb_body(x_hbm, tok_hbm, pos_hbm, out_hbm,
              pos_v, idx0, idx1, idx2, idx3, main0, main1, tail_v,
              isem0, isem1, isem2, isem3, gsem0, gsem1, tsem, osem0, osem1):
    wid = lax.axis_index("s") * NC + lax.axis_index("c")
    b0 = wid * BPW

    pltpu.sync_copy(pos_hbm, pos_v)

    idxs = (idx0, idx1, idx2, idx3)
    mains = (main0, main1)
    isems = (isem0, isem1, isem2, isem3)
    gsems = (gsem0, gsem1)
    osems = (osem0, osem1)

    def idx_src(c):
        return x_hbm.at[pl.ds((b0 + c) * SP, SP)]

    def start_idx(c, islot):
        pltpu.async_copy(idx_src(c), idxs[islot], isems[islot])

    def wait_idx(c, islot):
        pltpu.make_async_copy(idx_src(c), idxs[islot], isems[islot]).wait()

    def start_main_gather(c, islot, mslot):
        pltpu.async_copy(
            tok_hbm.at[idxs[islot].at[pl.ds(0, MAIN)]],
            mains[mslot].at[pl.ds(0, MAIN)], gsems[mslot])

    def wait_main_gather(c, islot, mslot):
        pltpu.make_async_copy(
            tok_hbm.at[idxs[islot].at[pl.ds(0, MAIN)]],
            mains[mslot].at[pl.ds(0, MAIN)], gsems[mslot]).wait()

    def start_tail_gather(c, islot):
        pltpu.async_copy(
            tok_hbm.at[idxs[islot].at[pl.ds(MAIN, SP - MAIN)]],
            tail_v, tsem)

    def wait_tail_gather(c, islot):
        pltpu.make_async_copy(
            tok_hbm.at[idxs[islot].at[pl.ds(MAIN, SP - MAIN)]],
            tail_v, tsem).wait()

    def start_out(c, mslot):
        pltpu.async_copy(mains[mslot], out_hbm.at[b0 + c], osems[mslot])

    def wait_out(c, mslot):
        pltpu.make_async_copy(mains[mslot], out_hbm.at[b0 + c], osems[mslot]).wait()

    start_idx(0, 0)
    start_idx(1, 1)
    wait_idx(0, 0)
    start_main_gather(0, 0, 0)
    start_tail_gather(0, 0)

    def quad(p, carry):
        for sl_ in range(4):
            c = 4 * p + sl_
            islot = sl_
            nislot = (sl_ + 1) % 4
            mslot, nmslot = sl_ % 2, 1 - sl_ % 2

            @pl.when(c >= 1)
            def _():
                wait_out(c - 1, nmslot)

            @pl.when(c + 2 < BPW)
            def _():
                start_idx(c + 2, (sl_ + 2) % 4)

            @pl.when(c + 1 < BPW)
            def _():
                wait_idx(c + 1, nislot)
                start_main_gather(c + 1, nislot, nmslot)

            wait_main_gather(c, islot, mslot)
            wait_tail_gather(c, islot)

            main_v = mains[mslot]

            for t in range(TAIL):
                for j in range(D // LANES):
                    dsl = pl.ds(j * LANES, LANES)
                    main_v[MAIN + t, dsl] = tail_v[t, dsl] + pos_v[MAIN + t, dsl]

            @pl.when(c + 1 < BPW)
            def _():
                start_tail_gather(c + 1, nislot)

            @plsc.parallel_loop(0, MAIN, unroll=4)
            def _(r):
                for j in range(D // LANES):
                    dsl = pl.ds(j * LANES, LANES)
                    main_v[r, dsl] = main_v[r, dsl] + pos_v[r, dsl]

            start_out(c, mslot)
        return carry

    lax.fori_loop(0, BPW // 4, quad, 0)
    wait_out(BPW - 1, 1)


@functools.partial(
    pl.kernel,
    out_type=jax.ShapeDtypeStruct((B, S, D), jnp.float32),
    mesh=plsc.VectorSubcoreMesh(
        core_axis_name="c", subcore_axis_name="s", num_cores=NC, num_subcores=NS
    ),
    scratch_types=[
        pltpu.VMEM((S, D), jnp.float32),
        pltpu.VMEM((SP,), jnp.int32),
        pltpu.VMEM((SP,), jnp.int32),
        pltpu.VMEM((SP,), jnp.int32),
        pltpu.VMEM((SP,), jnp.int32),
        pltpu.VMEM((S, D), jnp.float32),
        pltpu.VMEM((S, D), jnp.float32),
        pltpu.VMEM((8, D), jnp.float32),
        pltpu.SemaphoreType.DMA,
        pltpu.SemaphoreType.DMA,
        pltpu.SemaphoreType.DMA,
        pltpu.SemaphoreType.DMA,
        pltpu.SemaphoreType.DMA,
        pltpu.SemaphoreType.DMA,
        pltpu.SemaphoreType.DMA,
        pltpu.SemaphoreType.DMA,
        pltpu.SemaphoreType.DMA,
    ],
)
def _emb(x_hbm, tok_hbm, pos_hbm, out_hbm, *rest):
    _emb_body(x_hbm, tok_hbm, pos_hbm, out_hbm, *rest)


def kernel(x, token_table, position_table):
    x_pad = jnp.pad(x.astype(jnp.int32), ((0, 0), (0, SP - S))).reshape(B * SP)
    return _emb(x_pad, token_table, position_table)


# speedup vs baseline: 1.1409x; 1.0950x over previous
import functools

import jax
import jax.numpy as jnp
from jax import lax
from jax.experimental import pallas as pl
from jax.experimental.pallas import tpu as pltpu
from jax.experimental.pallas import tpu_sc as plsc

B, S, D = 4096, 77, 512
N = B * S
NC, NS = 2, 16
NW = NC * NS
RPW = N // NW
K = 32
NCHUNK = RPW // K
LANES = 16


def _emb_body(x_hbm, tok_hbm, pos_hbm, out_hbm,
              idx_v, pos_v, rows0, rows1, rows2, rows3,
              gsem0, gsem1, gsem2, gsem3, osem0, osem1, osem2, osem3):
    wid = lax.axis_index("s") * NC + lax.axis_index("c")
    base = wid * RPW

    pltpu.sync_copy(x_hbm.at[pl.ds(base, RPW)], idx_v)
    pltpu.sync_copy(pos_hbm, pos_v)

    bufs = (rows0, rows1, rows2, rows3)
    gsems = (gsem0, gsem1, gsem2, gsem3)
    osems = (osem0, osem1, osem2, osem3)

    def start_gather(c, slot):
        pltpu.async_copy(
            tok_hbm.at[idx_v.at[pl.ds(c * K, K)]], bufs[slot], gsems[slot])

    def wait_gather(c, slot):
        pltpu.make_async_copy(
            tok_hbm.at[idx_v.at[pl.ds(c * K, K)]], bufs[slot], gsems[slot]).wait()

    def start_out(c, slot):
        pltpu.async_copy(
            bufs[slot], out_hbm.at[pl.ds(base + c * K, K)], osems[slot])

    def wait_out(c, slot):
        pltpu.make_async_copy(
            bufs[slot], out_hbm.at[pl.ds(base + c * K, K)], osems[slot]).wait()

    start_gather(0, 0)

    def quad(p, carry):
        for sl_ in range(4):
            c = 4 * p + sl_
            slot = sl_
            nslot = (sl_ + 1) % 4

            @pl.when(c >= 3)
            def _():
                wait_out(c - 3, nslot)

            @pl.when(c + 1 < NCHUNK)
            def _():
                start_gather(c + 1, nslot)

            wait_gather(c, slot)

            buf = bufs[slot]
            s0 = lax.rem(c * K, S)

            @plsc.parallel_loop(0, K, unroll=4)
            def _(r):
                sr = s0 + r
                s = lax.select(sr >= S, sr - S, sr)
                for j in range(D // LANES):
                    dsl = pl.ds(j * LANES, LANES)
                    buf[r, dsl] = buf[r, dsl] + pos_v[s, dsl]

            start_out(c, slot)
        return carry

    lax.fori_loop(0, NCHUNK // 4, quad, 0)
    wait_out(NCHUNK - 3, (NCHUNK - 3) % 4)
    wait_out(NCHUNK - 2, (NCHUNK - 2) % 4)
    wait_out(NCHUNK - 1, (NCHUNK - 1) % 4)


@functools.partial(
    pl.kernel,
    out_type=jax.ShapeDtypeStruct((N, D), jnp.float32),
    mesh=plsc.VectorSubcoreMesh(
        core_axis_name="c", subcore_axis_name="s", num_cores=NC, num_subcores=NS
    ),
    scratch_types=[
        pltpu.VMEM((RPW,), jnp.int32),
        pltpu.VMEM((S, D), jnp.float32),
        pltpu.VMEM((K, D), jnp.float32),
        pltpu.VMEM((K, D), jnp.float32),
        pltpu.VMEM((K, D), jnp.float32),
        pltpu.VMEM((K, D), jnp.float32),
        pltpu.SemaphoreType.DMA,
        pltpu.SemaphoreType.DMA,
        pltpu.SemaphoreType.DMA,
        pltpu.SemaphoreType.DMA,
        pltpu.SemaphoreType.DMA,
        pltpu.SemaphoreType.DMA,
        pltpu.SemaphoreType.DMA,
        pltpu.SemaphoreType.DMA,
    ],
)
def _emb(x_hbm, tok_hbm, pos_hbm, out_hbm, *rest):
    _emb_body(x_hbm, tok_hbm, pos_hbm, out_hbm, *rest)


def kernel(x, token_table, position_table):
    out = _emb(x.astype(jnp.int32).reshape(N), token_table, position_table)
    return out.reshape(B, S, D)
